# baseline (speedup 1.0000x reference)
import functools

import jax
import jax.numpy as jnp
import numpy as np
from jax.experimental import pallas as pl

N = 20000
C = 80
PRE_NMS_THRESH = 0.05
PRE_NMS_TOP_N = 1000
NMS_THRESH = 0.6
POST_TOP_N = 100
WEIGHTS = (10.0, 10.0, 5.0, 5.0)
BBOX_XFORM_CLIP = float(np.log(1000.0 / 16.0))


def _fuse_kernel(cls_ref, iou_ref, out_ref):
    s = jax.nn.sigmoid(cls_ref[...]) * jax.nn.sigmoid(iou_ref[...])
    s = jnp.sqrt(s)
    out_ref[...] = jnp.where(s > PRE_NMS_THRESH, s, 0.0)


def _fused_scores(box_cls, iou_pred):
    blk = 400
    return pl.pallas_call(
        _fuse_kernel,
        grid=(N // blk,),
        in_specs=[
            pl.BlockSpec((blk, C), lambda i: (i, 0)),
            pl.BlockSpec((blk, 1), lambda i: (i, 0)),
        ],
        out_specs=pl.BlockSpec((blk, C), lambda i: (i, 0)),
        out_shape=jax.ShapeDtypeStruct((N, C), jnp.float32),
    )(box_cls, iou_pred.reshape(N, 1))


def _decode(deltas, anchors):
    wx, wy, ww, wh = WEIGHTS
    widths = anchors[:, 2] - anchors[:, 0]
    heights = anchors[:, 3] - anchors[:, 1]
    ctr_x = anchors[:, 0] + 0.5 * widths
    ctr_y = anchors[:, 1] + 0.5 * heights
    dx = deltas[:, 0] / wx
    dy = deltas[:, 1] / wy
    dw = jnp.minimum(deltas[:, 2] / ww, BBOX_XFORM_CLIP)
    dh = jnp.minimum(deltas[:, 3] / wh, BBOX_XFORM_CLIP)
    pcx = dx * widths + ctr_x
    pcy = dy * heights + ctr_y
    pw = jnp.exp(dw) * widths
    ph = jnp.exp(dh) * heights
    return jnp.stack([pcx - 0.5 * pw, pcy - 0.5 * ph, pcx + 0.5 * pw, pcy + 0.5 * ph], axis=1)


def kernel(box_cls, box_regression, iou_pred, anchors):
    scores = _fused_scores(box_cls, iou_pred)
    flat = scores.reshape(-1)
    top_scores, top_idx = jax.lax.top_k(flat, PRE_NMS_TOP_N)
    box_idx = top_idx // C
    labels = top_idx % C
    boxes = _decode(box_regression[box_idx], anchors[box_idx])
    off = labels.astype(jnp.float32) * 10000.0
    boxes_off = boxes + off[:, None]
    area = (boxes_off[:, 2] - boxes_off[:, 0]) * (boxes_off[:, 3] - boxes_off[:, 1])
    lt = jnp.maximum(boxes_off[:, None, :2], boxes_off[None, :, :2])
    rb = jnp.minimum(boxes_off[:, None, 2:], boxes_off[None, :, 2:])
    whs = jnp.clip(rb - lt, 0.0)
    inter = whs[..., 0] * whs[..., 1]
    iou = inter / (area[:, None] + area[None, :] - inter + 1e-7)
    K = PRE_NMS_TOP_N
    idx = jnp.arange(K)

    def body(i, keep):
        sup = (iou[i] > NMS_THRESH) & (idx > i) & keep[i]
        return keep & (~sup)

    keep = jax.lax.fori_loop(0, K, body, jnp.ones((K,), dtype=bool))
    final_scores = jnp.where(keep, top_scores, 0.0)
    sel_scores, sel = jax.lax.top_k(final_scores, POST_TOP_N)
    out = jnp.concatenate([boxes[sel], sel_scores[:, None]], axis=1)
    return out


# speedup vs baseline: 1.4547x; 1.4547x over previous
import functools

import jax
import jax.numpy as jnp
import numpy as np
from jax.experimental import pallas as pl

N = 20000
C = 80
PRE_NMS_THRESH = 0.05
PRE_NMS_TOP_N = 1000
NMS_THRESH = 0.6
POST_TOP_N = 100
WEIGHTS = (10.0, 10.0, 5.0, 5.0)
BBOX_XFORM_CLIP = float(np.log(1000.0 / 16.0))


def _fuse_kernel(cls_ref, iou_ref, out_ref):
    s = jax.nn.sigmoid(cls_ref[...]) * jax.nn.sigmoid(iou_ref[...])
    s = jnp.sqrt(s)
    out_ref[...] = jnp.where(s > PRE_NMS_THRESH, s, 0.0)


def _fused_scores(box_cls, iou_pred):
    blk = 400
    return pl.pallas_call(
        _fuse_kernel,
        grid=(N // blk,),
        in_specs=[
            pl.BlockSpec((blk, C), lambda i: (i, 0)),
            pl.BlockSpec((blk, 1), lambda i: (i, 0)),
        ],
        out_specs=pl.BlockSpec((blk, C), lambda i: (i, 0)),
        out_shape=jax.ShapeDtypeStruct((N, C), jnp.float32),
    )(box_cls, iou_pred.reshape(N, 1))


KPAD = 1024


def _finalize_kernel(scores_ref, labels_ref, deltas_ref, anchors_ref,
                     out_boxes_ref, out_scores_ref, boxes_ref, iou_ref):
    wx, wy, ww, wh = WEIGHTS
    a = anchors_ref[...]
    d = deltas_ref[...]
    widths = a[:, 2:3] - a[:, 0:1]
    heights = a[:, 3:4] - a[:, 1:2]
    ctr_x = a[:, 0:1] + 0.5 * widths
    ctr_y = a[:, 1:2] + 0.5 * heights
    dx = d[:, 0:1] / wx
    dy = d[:, 1:2] / wy
    dw = jnp.minimum(d[:, 2:3] / ww, BBOX_XFORM_CLIP)
    dh = jnp.minimum(d[:, 3:4] / wh, BBOX_XFORM_CLIP)
    pcx = dx * widths + ctr_x
    pcy = dy * heights + ctr_y
    pw = jnp.exp(dw) * widths
    ph = jnp.exp(dh) * heights
    x0 = pcx - 0.5 * pw
    y0 = pcy - 0.5 * ph
    x1 = pcx + 0.5 * pw
    y1 = pcy + 0.5 * ph
    boxes_ref[...] = jnp.concatenate([x0, y0, x1, y1], axis=1)

    off = labels_ref[...] * 10000.0
    ox0 = x0 + off
    oy0 = y0 + off
    ox1 = x1 + off
    oy1 = y1 + off
    area = (ox1 - ox0) * (oy1 - oy0)
    ltx = jnp.maximum(ox0, ox0.reshape(1, KPAD))
    lty = jnp.maximum(oy0, oy0.reshape(1, KPAD))
    rbx = jnp.minimum(ox1, ox1.reshape(1, KPAD))
    rby = jnp.minimum(oy1, oy1.reshape(1, KPAD))
    iw = jnp.maximum(rbx - ltx, 0.0)
    ih = jnp.maximum(rby - lty, 0.0)
    inter = iw * ih
    iou_ref[...] = inter / (area + area.reshape(1, KPAD) - inter + 1e-7)

    iota = jax.lax.broadcasted_iota(jnp.int32, (1, KPAD), 1)

    def nms_body(i, keep):
        row = iou_ref[pl.ds(i, 1), :]
        k_i = jnp.sum(jnp.where(iota == i, keep, 0.0))
        sup = (row > NMS_THRESH) & (iota > i) & (k_i > 0.5)
        return keep * (1.0 - sup.astype(jnp.float32))

    keep = jax.lax.fori_loop(0, PRE_NMS_TOP_N, nms_body,
                             jnp.ones((1, KPAD), jnp.float32))
    final = keep * scores_ref[...]

    def sel_body(k, fin):
        v = jnp.max(fin)
        am = jnp.min(jnp.where(fin == v, iota, KPAD))
        out_boxes_ref[pl.ds(k, 1), :] = boxes_ref[pl.ds(am, 1), :]
        out_scores_ref[pl.ds(k, 1), :] = v.reshape(1, 1)
        return jnp.where(iota == am, -1.0, fin)

    jax.lax.fori_loop(0, POST_TOP_N, sel_body, final)


def _finalize(top_scores, labels, deltas_g, anchors_g):
    from jax.experimental.pallas import tpu as pltpu
    npad = KPAD - PRE_NMS_TOP_N
    scores_p = jnp.concatenate([top_scores, jnp.zeros((npad,), jnp.float32)]).reshape(1, KPAD)
    labels_p = jnp.concatenate([labels.astype(jnp.float32),
                                jnp.full((npad,), 200.0, jnp.float32)]).reshape(KPAD, 1)
    deltas_p = jnp.concatenate([deltas_g, jnp.zeros((npad, 4), jnp.float32)], axis=0)
    anchors_p = jnp.concatenate([anchors_g, jnp.zeros((npad, 4), jnp.float32)], axis=0)
    boxes100, scores100 = pl.pallas_call(
        _finalize_kernel,
        out_shape=[
            jax.ShapeDtypeStruct((POST_TOP_N, 4), jnp.float32),
            jax.ShapeDtypeStruct((POST_TOP_N, 1), jnp.float32),
        ],
        scratch_shapes=[
            pltpu.VMEM((KPAD, 4), jnp.float32),
            pltpu.VMEM((KPAD, KPAD), jnp.float32),
        ],
    )(scores_p, labels_p, deltas_p, anchors_p)
    return jnp.concatenate([boxes100, scores100], axis=1)


def kernel(box_cls, box_regression, iou_pred, anchors):
    scores = _fused_scores(box_cls, iou_pred)
    flat = scores.reshape(-1)
    top_scores, top_idx = jax.lax.top_k(flat, PRE_NMS_TOP_N)
    box_idx = top_idx // C
    labels = top_idx % C
    deltas_g = box_regression[box_idx]
    anchors_g = anchors[box_idx]
    return _finalize(top_scores, labels, deltas_g, anchors_g)


# speedup vs baseline: 3.1654x; 2.1759x over previous
import jax
import jax.numpy as jnp
import numpy as np
from jax.experimental import pallas as pl
from jax.experimental.pallas import tpu as pltpu

N = 20000
C = 80
PRE_NMS_THRESH = 0.05
PRE_NMS_TOP_N = 1000
NMS_THRESH = 0.6
POST_TOP_N = 100
WEIGHTS = (10.0, 10.0, 5.0, 5.0)
BBOX_XFORM_CLIP = float(np.log(1000.0 / 16.0))

KPAD = 1024
NCHUNK = 25
CHUNK = N // NCHUNK
CPAD = 32
QROWS = 128
BIG = 1 << 30


def _paa_kernel(cls_ref, regan_ref, iou_in_ref,
                out_boxes_ref, out_scores_ref,
                scores_ref, rowmax_ref, chunkmax_ref,
                sel_scores_ref, sel_labels_ref, sel_regan_ref,
                boxes_ref, iou_ref):
    cms = []
    for k in range(NCHUNK):
        iou_col = iou_in_ref[k:k + 1, :].reshape(CHUNK, 1)
        s = jnp.sqrt(jax.nn.sigmoid(cls_ref[k * CHUNK:(k + 1) * CHUNK, :])
                     * jax.nn.sigmoid(iou_col))
        s = jnp.where(s > PRE_NMS_THRESH, s, 0.0)
        scores_ref[k * CHUNK:(k + 1) * CHUNK, :] = s
        rm = jnp.max(s, axis=1).reshape(1, CHUNK)
        rowmax_ref[k:k + 1, :] = rm
        cms.append(jnp.max(rm))
    cms += [jnp.float32(-1.0)] * (CPAD - NCHUNK)
    chunkmax_ref[...] = jnp.stack(cms).reshape(CPAD, 1)

    sel_scores_ref[...] = jnp.zeros((KPAD, 1), jnp.float32)
    sel_labels_ref[...] = jnp.full((KPAD, 1), 200.0, jnp.float32)
    sel_regan_ref[...] = jnp.zeros((KPAD, 8), jnp.float32)

    lane_c = jax.lax.broadcasted_iota(jnp.int32, (1, C), 1)
    lane_chunk = jax.lax.broadcasted_iota(jnp.int32, (1, CHUNK), 1)
    iota_cpad = jax.lax.broadcasted_iota(jnp.int32, (CPAD, 1), 0)

    def ext_cond(state):
        n, v = state
        return (n < PRE_NMS_TOP_N) & (v > 0.0)

    def ext_body(state):
        n, v = state
        cm = chunkmax_ref[...]
        ck = jnp.min(jnp.where(cm == v, iota_cpad, BIG))
        rrow = rowmax_ref[pl.ds(ck, 1), :]
        rloc = jnp.min(jnp.where(rrow == v, lane_chunk, BIG))
        r = ck * CHUNK + rloc
        row = scores_ref[pl.ds(r, 1), :]
        col = jnp.min(jnp.where(row == v, lane_c, BIG))
        sel_scores_ref[pl.ds(n, 1), :] = v.reshape(1, 1)
        sel_labels_ref[pl.ds(n, 1), :] = col.astype(jnp.float32).reshape(1, 1)
        sel_regan_ref[pl.ds(n, 1), :] = regan_ref[pl.ds(r, 1), :]
        new_row = jnp.where(lane_c == col, 0.0, row)
        scores_ref[pl.ds(r, 1), :] = new_row
        rrow_new = jnp.where(lane_chunk == rloc, jnp.max(new_row), rrow)
        rowmax_ref[pl.ds(ck, 1), :] = rrow_new
        cm_new = jnp.where(iota_cpad == ck, jnp.max(rrow_new), cm)
        chunkmax_ref[...] = cm_new
        return n + 1, jnp.max(cm_new)

    v0 = jnp.max(chunkmax_ref[...])
    jax.lax.while_loop(ext_cond, ext_body, (jnp.int32(0), v0))

    wx, wy, ww, wh = WEIGHTS
    ra = sel_regan_ref[...]
    widths = ra[:, 6:7] - ra[:, 4:5]
    heights = ra[:, 7:8] - ra[:, 5:6]
    ctr_x = ra[:, 4:5] + 0.5 * widths
    ctr_y = ra[:, 5:6] + 0.5 * heights
    dx = ra[:, 0:1] / wx
    dy = ra[:, 1:2] / wy
    dw = jnp.minimum(ra[:, 2:3] / ww, BBOX_XFORM_CLIP)
    dh = jnp.minimum(ra[:, 3:4] / wh, BBOX_XFORM_CLIP)
    pcx = dx * widths + ctr_x
    pcy = dy * heights + ctr_y
    pw = jnp.exp(dw) * widths
    ph = jnp.exp(dh) * heights
    x0 = pcx - 0.5 * pw
    y0 = pcy - 0.5 * ph
    x1 = pcx + 0.5 * pw
    y1 = pcy + 0.5 * ph
    boxes_ref[...] = jnp.concatenate([x0, y0, x1, y1], axis=1)

    off = sel_labels_ref[...] * 10000.0
    ox0 = x0 + off
    oy0 = y0 + off
    ox1 = x1 + off
    oy1 = y1 + off
    area = (ox1 - ox0) * (oy1 - oy0)
    ox0r = ox0.reshape(1, KPAD)
    oy0r = oy0.reshape(1, KPAD)
    ox1r = ox1.reshape(1, KPAD)
    oy1r = oy1.reshape(1, KPAD)
    arear = area.reshape(1, KPAD)
    for q in range(KPAD // QROWS):
        sl = slice(q * QROWS, (q + 1) * QROWS)
        iw = jnp.maximum(jnp.minimum(ox1[sl], ox1r) - jnp.maximum(ox0[sl], ox0r), 0.0)
        ih = jnp.maximum(jnp.minimum(oy1[sl], oy1r) - jnp.maximum(oy0[sl], oy0r), 0.0)
        inter = iw * ih
        iou_ref[sl, :] = inter / (area[sl] + arear - inter + 1e-7)

    kiota = jax.lax.broadcasted_iota(jnp.int32, (1, KPAD), 1)

    def nms_body(i, keep):
        row = iou_ref[pl.ds(i, 1), :]
        k_i = jnp.sum(jnp.where(kiota == i, keep, 0.0))
        sup = (row > NMS_THRESH) & (kiota > i) & (k_i > 0.5)
        return keep * (1.0 - sup.astype(jnp.float32))

    keep = jax.lax.fori_loop(0, PRE_NMS_TOP_N, nms_body,
                             jnp.ones((1, KPAD), jnp.float32))
    final = keep * sel_scores_ref[...].reshape(1, KPAD)

    def sel_body(k, fin):
        v = jnp.max(fin)
        am = jnp.min(jnp.where(fin == v, kiota, BIG))
        out_boxes_ref[pl.ds(k, 1), :] = boxes_ref[pl.ds(am, 1), :]
        out_scores_ref[pl.ds(k, 1), :] = v.reshape(1, 1)
        return jnp.where(kiota == am, -1.0, fin)

    jax.lax.fori_loop(0, POST_TOP_N, sel_body, final)


def kernel(box_cls, box_regression, iou_pred, anchors):
    regan = jnp.concatenate([box_regression, anchors], axis=1)
    boxes100, scores100 = pl.pallas_call(
        _paa_kernel,
        out_shape=[
            jax.ShapeDtypeStruct((POST_TOP_N, 4), jnp.float32),
            jax.ShapeDtypeStruct((POST_TOP_N, 1), jnp.float32),
        ],
        scratch_shapes=[
            pltpu.VMEM((N, C), jnp.float32),
            pltpu.VMEM((NCHUNK, CHUNK), jnp.float32),
            pltpu.VMEM((CPAD, 1), jnp.float32),
            pltpu.VMEM((KPAD, 1), jnp.float32),
            pltpu.VMEM((KPAD, 1), jnp.float32),
            pltpu.VMEM((KPAD, 8), jnp.float32),
            pltpu.VMEM((KPAD, 4), jnp.float32),
            pltpu.VMEM((KPAD, KPAD), jnp.float32),
        ],
    )(box_cls, regan, iou_pred.reshape(NCHUNK, CHUNK))
    return jnp.concatenate([boxes100, scores100], axis=1)


# speedup vs baseline: 3.1971x; 1.0100x over previous
import jax
import jax.numpy as jnp
import numpy as np
from jax.experimental import pallas as pl
from jax.experimental.pallas import tpu as pltpu

N = 20000
C = 80
PRE_NMS_THRESH = 0.05
PRE_NMS_TOP_N = 1000
NMS_THRESH = 0.6
POST_TOP_N = 100
WEIGHTS = (10.0, 10.0, 5.0, 5.0)
BBOX_XFORM_CLIP = float(np.log(1000.0 / 16.0))

KPAD = 1024
NCHUNK = 40
CHUNK = N // NCHUNK
CPAD = 64
QROWS = 128
NMSB = 8
BIG = 1 << 30


def _paa_kernel(cls_ref, regan_ref, iou_in_ref,
                out_boxes_ref, out_scores_ref,
                scores_ref, rowmax_ref, sel_ref, boxes_ref, iou_ref):
    cms = []
    for k in range(NCHUNK):
        iou_col = iou_in_ref[k:k + 1, :].reshape(CHUNK, 1)
        s = jnp.sqrt(jax.nn.sigmoid(cls_ref[k * CHUNK:(k + 1) * CHUNK, :])
                     * jax.nn.sigmoid(iou_col))
        s = jnp.where(s > PRE_NMS_THRESH, s, 0.0)
        scores_ref[k * CHUNK:(k + 1) * CHUNK, :] = s
        rm = jnp.max(s, axis=1).reshape(1, CHUNK)
        rowmax_ref[k:k + 1, :] = rm
        cms.append(jnp.max(rm))
    cms += [jnp.float32(-1.0)] * (CPAD - NCHUNK)
    cm0 = jnp.stack(cms).reshape(1, CPAD)

    lane_c = jax.lax.broadcasted_iota(jnp.int32, (1, C), 1)
    lane_chunk = jax.lax.broadcasted_iota(jnp.int32, (1, CHUNK), 1)
    lane_cm = jax.lax.broadcasted_iota(jnp.int32, (1, CPAD), 1)
    lane_8 = jax.lax.broadcasted_iota(jnp.int32, (1, 8), 1)
    lane_16 = jax.lax.broadcasted_iota(jnp.int32, (1, 16), 1)

    sel_ref[...] = jnp.where(
        jax.lax.broadcasted_iota(jnp.int32, (KPAD, 16), 1) == 9, 200.0, 0.0)

    def ext_cond(state):
        n, v, _ = state
        return (n < PRE_NMS_TOP_N) & (v > 0.0)

    def ext_body(state):
        n, v, cm = state
        ck = jnp.min(jnp.where(cm == v, lane_cm, BIG))
        rrow = rowmax_ref[pl.ds(ck, 1), :]
        rloc = jnp.min(jnp.where(rrow == v, lane_chunk, BIG))
        r = ck * CHUNK + rloc
        row = scores_ref[pl.ds(r, 1), :]
        col = jnp.min(jnp.where(row == v, lane_c, BIG))
        regrow = regan_ref[pl.ds(r, 1), :]
        meta = jnp.where(lane_8 == 0, v,
                         jnp.where(lane_8 == 1, col.astype(jnp.float32), 0.0))
        sel_ref[pl.ds(n, 1), :] = jnp.concatenate([regrow, meta], axis=1)
        new_row = jnp.where(lane_c == col, 0.0, row)
        scores_ref[pl.ds(r, 1), :] = new_row
        rrow_new = jnp.where(lane_chunk == rloc, jnp.max(new_row), rrow)
        rowmax_ref[pl.ds(ck, 1), :] = rrow_new
        cm_new = jnp.where(lane_cm == ck, jnp.max(rrow_new), cm)
        return n + 1, jnp.max(cm_new), cm_new

    jax.lax.while_loop(ext_cond, ext_body, (jnp.int32(0), jnp.max(cm0), cm0))

    wx, wy, ww, wh = WEIGHTS
    ra = sel_ref[...]
    widths = ra[:, 6:7] - ra[:, 4:5]
    heights = ra[:, 7:8] - ra[:, 5:6]
    ctr_x = ra[:, 4:5] + 0.5 * widths
    ctr_y = ra[:, 5:6] + 0.5 * heights
    dx = ra[:, 0:1] / wx
    dy = ra[:, 1:2] / wy
    dw = jnp.minimum(ra[:, 2:3] / ww, BBOX_XFORM_CLIP)
    dh = jnp.minimum(ra[:, 3:4] / wh, BBOX_XFORM_CLIP)
    pcx = dx * widths + ctr_x
    pcy = dy * heights + ctr_y
    pw = jnp.exp(dw) * widths
    ph = jnp.exp(dh) * heights
    x0 = pcx - 0.5 * pw
    y0 = pcy - 0.5 * ph
    x1 = pcx + 0.5 * pw
    y1 = pcy + 0.5 * ph
    boxes_ref[...] = jnp.concatenate([x0, y0, x1, y1], axis=1)

    off = ra[:, 9:10] * 10000.0
    ox0 = x0 + off
    oy0 = y0 + off
    ox1 = x1 + off
    oy1 = y1 + off
    area = (ox1 - ox0) * (oy1 - oy0)
    ox0r = ox0.reshape(1, KPAD)
    oy0r = oy0.reshape(1, KPAD)
    ox1r = ox1.reshape(1, KPAD)
    oy1r = oy1.reshape(1, KPAD)
    arear = area.reshape(1, KPAD)
    for q in range(KPAD // QROWS):
        sl = slice(q * QROWS, (q + 1) * QROWS)
        iw = jnp.maximum(jnp.minimum(ox1[sl], ox1r) - jnp.maximum(ox0[sl], ox0r), 0.0)
        ih = jnp.maximum(jnp.minimum(oy1[sl], oy1r) - jnp.maximum(oy0[sl], oy0r), 0.0)
        inter = iw * ih
        iou_ref[sl, :] = inter / (area[sl] + arear - inter + 1e-7)

    kiota = jax.lax.broadcasted_iota(jnp.int32, (1, KPAD), 1)

    def nms_body(i, keep):
        rows = iou_ref[pl.ds(i * NMSB, NMSB), :]
        for j in range(NMSB):
            idx = i * NMSB + j
            row = rows[j:j + 1, :]
            k_i = jnp.sum(jnp.where(kiota == idx, keep, 0.0))
            sup = (row > NMS_THRESH) & (kiota > idx) & (k_i > 0.5)
            keep = keep * (1.0 - sup.astype(jnp.float32))
        return keep

    keep = jax.lax.fori_loop(0, PRE_NMS_TOP_N // NMSB, nms_body,
                             jnp.ones((1, KPAD), jnp.float32))
    final = keep * sel_ref[:, 8:9].reshape(1, KPAD)

    def sel_body(k, fin):
        v = jnp.max(fin)
        am = jnp.min(jnp.where(fin == v, kiota, BIG))
        out_boxes_ref[pl.ds(k, 1), :] = boxes_ref[pl.ds(am, 1), :]
        out_scores_ref[pl.ds(k, 1), :] = v.reshape(1, 1)
        return jnp.where(kiota == am, -1.0, fin)

    jax.lax.fori_loop(0, POST_TOP_N, sel_body, final)


def kernel(box_cls, box_regression, iou_pred, anchors):
    regan = jnp.concatenate([box_regression, anchors], axis=1)
    boxes100, scores100 = pl.pallas_call(
        _paa_kernel,
        out_shape=[
            jax.ShapeDtypeStruct((POST_TOP_N, 4), jnp.float32),
            jax.ShapeDtypeStruct((POST_TOP_N, 1), jnp.float32),
        ],
        scratch_shapes=[
            pltpu.VMEM((N, C), jnp.float32),
            pltpu.VMEM((NCHUNK, CHUNK), jnp.float32),
            pltpu.VMEM((KPAD, 16), jnp.float32),
            pltpu.VMEM((KPAD, 4), jnp.float32),
            pltpu.VMEM((KPAD, KPAD), jnp.float32),
        ],
    )(box_cls, regan, iou_pred.reshape(NCHUNK, CHUNK))
    return jnp.concatenate([boxes100, scores100], axis=1)


# speedup vs baseline: 3.1979x; 1.0002x over previous
import jax
import jax.numpy as jnp
import numpy as np
from jax.experimental import pallas as pl
from jax.experimental.pallas import tpu as pltpu

N = 20000
C = 80
PRE_NMS_THRESH = 0.05
PRE_NMS_TOP_N = 1000
NMS_THRESH = 0.6
POST_TOP_N = 100
WEIGHTS = (10.0, 10.0, 5.0, 5.0)
BBOX_XFORM_CLIP = float(np.log(1000.0 / 16.0))

KPAD = 1024
NCHUNK = 40
CHUNK = N // NCHUNK
CPAD = 64
QROWS = 128
NMSB = 8
BIG = 1 << 30


def _paa_kernel(cls_ref, regan_ref, iou_in_ref,
                out_boxes_ref, out_scores_ref,
                scores_ref, rowmax_ref, sel_ref, boxes_ref, iou_ref):
    cms = []
    for k in range(NCHUNK):
        iou_col = iou_in_ref[k:k + 1, :].reshape(CHUNK, 1)
        s = jnp.sqrt(jax.nn.sigmoid(cls_ref[k * CHUNK:(k + 1) * CHUNK, :])
                     * jax.nn.sigmoid(iou_col))
        s = jnp.where(s > PRE_NMS_THRESH, s, 0.0)
        scores_ref[k * CHUNK:(k + 1) * CHUNK, :] = s
        rm = jnp.max(s, axis=1).reshape(1, CHUNK)
        rowmax_ref[k:k + 1, :] = rm
        cms.append(jnp.max(rm))
    cms += [jnp.float32(-1.0)] * (CPAD - NCHUNK)
    cm0 = jnp.stack(cms).reshape(1, CPAD)

    lane_c = jax.lax.broadcasted_iota(jnp.int32, (1, C), 1)
    lane_chunk = jax.lax.broadcasted_iota(jnp.int32, (1, CHUNK), 1)
    lane_cm = jax.lax.broadcasted_iota(jnp.int32, (1, CPAD), 1)
    lane_8 = jax.lax.broadcasted_iota(jnp.int32, (1, 8), 1)

    sel_ref[...] = jnp.where(
        jax.lax.broadcasted_iota(jnp.int32, (KPAD, 16), 1) == 9, 200.0, 0.0)

    def ext_cond(state):
        n, v, _ = state
        return (n < PRE_NMS_TOP_N) & (v > 0.0)

    def ext_body(state):
        n, v, cm = state
        ck = jnp.min(jnp.where(cm == v, lane_cm, BIG))
        rrow = rowmax_ref[pl.ds(ck, 1), :]
        rloc = jnp.min(jnp.where(rrow == v, lane_chunk, BIG))
        r = ck * CHUNK + rloc
        row = scores_ref[pl.ds(r, 1), :]
        col = jnp.min(jnp.where(row == v, lane_c, BIG))
        regrow = regan_ref[pl.ds(r, 1), :]
        meta = jnp.where(lane_8 == 0, v,
                         jnp.where(lane_8 == 1, col.astype(jnp.float32), 0.0))
        sel_ref[pl.ds(n, 1), :] = jnp.concatenate([regrow, meta], axis=1)
        new_row = jnp.where(lane_c == col, 0.0, row)
        scores_ref[pl.ds(r, 1), :] = new_row
        rrow_new = jnp.where(lane_chunk == rloc, jnp.max(new_row), rrow)
        rowmax_ref[pl.ds(ck, 1), :] = rrow_new
        cm_new = jnp.where(lane_cm == ck, jnp.max(rrow_new), cm)
        return n + 1, jnp.max(cm_new), cm_new

    jax.lax.while_loop(ext_cond, ext_body, (jnp.int32(0), jnp.max(cm0), cm0))

    wx, wy, ww, wh = WEIGHTS
    ra = sel_ref[...]
    widths = ra[:, 6:7] - ra[:, 4:5]
    heights = ra[:, 7:8] - ra[:, 5:6]
    ctr_x = ra[:, 4:5] + 0.5 * widths
    ctr_y = ra[:, 5:6] + 0.5 * heights
    dx = ra[:, 0:1] / wx
    dy = ra[:, 1:2] / wy
    dw = jnp.minimum(ra[:, 2:3] / ww, BBOX_XFORM_CLIP)
    dh = jnp.minimum(ra[:, 3:4] / wh, BBOX_XFORM_CLIP)
    pcx = dx * widths + ctr_x
    pcy = dy * heights + ctr_y
    pw = jnp.exp(dw) * widths
    ph = jnp.exp(dh) * heights
    x0 = pcx - 0.5 * pw
    y0 = pcy - 0.5 * ph
    x1 = pcx + 0.5 * pw
    y1 = pcy + 0.5 * ph
    boxes_ref[...] = jnp.concatenate([x0, y0, x1, y1], axis=1)

    off = ra[:, 9:10] * 10000.0
    ox0 = x0 + off
    oy0 = y0 + off
    ox1 = x1 + off
    oy1 = y1 + off
    area = (ox1 - ox0) * (oy1 - oy0)
    ox0r = ox0.reshape(1, KPAD)
    oy0r = oy0.reshape(1, KPAD)
    ox1r = ox1.reshape(1, KPAD)
    oy1r = oy1.reshape(1, KPAD)
    arear = area.reshape(1, KPAD)
    for q in range(KPAD // QROWS):
        sl = slice(q * QROWS, (q + 1) * QROWS)
        iw = jnp.maximum(jnp.minimum(ox1[sl], ox1r) - jnp.maximum(ox0[sl], ox0r), 0.0)
        ih = jnp.maximum(jnp.minimum(oy1[sl], oy1r) - jnp.maximum(oy0[sl], oy0r), 0.0)
        inter = iw * ih
        iou_ref[sl, :] = inter / (area[sl] + arear - inter + 1e-7)

    kiota = jax.lax.broadcasted_iota(jnp.int32, (1, KPAD), 1)

    def nms_body(i, keep):
        rows = iou_ref[pl.ds(i * NMSB, NMSB), :]
        for j in range(NMSB):
            idx = i * NMSB + j
            row = rows[j:j + 1, :]
            k_i = jnp.sum(jnp.where(kiota == idx, keep, 0.0))
            sup = (row > NMS_THRESH) & (kiota > idx) & (k_i > 0.5)
            keep = keep * (1.0 - sup.astype(jnp.float32))
        return keep

    keep = jax.lax.fori_loop(0, PRE_NMS_TOP_N // NMSB, nms_body,
                             jnp.ones((1, KPAD), jnp.float32))
    final = keep * sel_ref[:, 8:9].reshape(1, KPAD)

    def sel_body(k, fin):
        v = jnp.max(fin)
        am = jnp.min(jnp.where(fin == v, kiota, BIG))
        out_boxes_ref[pl.ds(k, 1), :] = boxes_ref[pl.ds(am, 1), :]
        out_scores_ref[pl.ds(k, 1), :] = v.reshape(1, 1)
        return jnp.where(kiota == am, -1.0, fin)

    jax.lax.fori_loop(0, POST_TOP_N, sel_body, final)


def kernel(box_cls, box_regression, iou_pred, anchors):
    regan = jnp.concatenate([box_regression, anchors], axis=1)
    boxes100, scores100 = pl.pallas_call(
        _paa_kernel,
        out_shape=[
            jax.ShapeDtypeStruct((POST_TOP_N, 4), jnp.float32),
            jax.ShapeDtypeStruct((POST_TOP_N, 1), jnp.float32),
        ],
        scratch_shapes=[
            pltpu.VMEM((N, C), jnp.float32),
            pltpu.VMEM((NCHUNK, CHUNK), jnp.float32),
            pltpu.VMEM((KPAD, 16), jnp.float32),
            pltpu.VMEM((KPAD, 4), jnp.float32),
            pltpu.VMEM((KPAD, KPAD), jnp.float32),
        ],
    )(box_cls, regan, iou_pred.reshape(NCHUNK, CHUNK))
    return jnp.concatenate([boxes100, scores100], axis=1)


# speedup vs baseline: 4.3141x; 1.3491x over previous
import jax
import jax.numpy as jnp
import numpy as np
from jax.experimental import pallas as pl
from jax.experimental.pallas import tpu as pltpu

N = 20000
C = 80
PRE_NMS_THRESH = 0.05
PRE_NMS_TOP_N = 1000
NMS_THRESH = 0.6
POST_TOP_N = 100
WEIGHTS = (10.0, 10.0, 5.0, 5.0)
BBOX_XFORM_CLIP = float(np.log(1000.0 / 16.0))

KPAD = 1024
NCHUNK = 40
CHUNK = N // NCHUNK
CPAD = 64
QROWS = 128
NMSB = 8
BIG = 1 << 30


def _paa_kernel(cls_ref, regan_ref, iou_in_ref,
                out_boxes_ref, out_scores_ref,
                scores_ref, rowmax_ref, sel_ref, boxes_ref, iou_ref):
    cms = []
    for k in range(NCHUNK):
        iou_col = iou_in_ref[k:k + 1, :].reshape(CHUNK, 1)
        s = jnp.sqrt(jax.nn.sigmoid(cls_ref[k * CHUNK:(k + 1) * CHUNK, :])
                     * jax.nn.sigmoid(iou_col))
        s = jnp.where(s > PRE_NMS_THRESH, s, 0.0)
        scores_ref[k * CHUNK:(k + 1) * CHUNK, :] = s
        rm = jnp.max(s, axis=1).reshape(1, CHUNK)
        rowmax_ref[k:k + 1, :] = rm

    lane_c = jax.lax.broadcasted_iota(jnp.int32, (1, C), 1)
    lane_8 = jax.lax.broadcasted_iota(jnp.int32, (1, 8), 1)
    flat2d = (jax.lax.broadcasted_iota(jnp.int32, (NCHUNK, CHUNK), 0) * CHUNK
              + jax.lax.broadcasted_iota(jnp.int32, (NCHUNK, CHUNK), 1))

    sel_ref[...] = jnp.where(
        jax.lax.broadcasted_iota(jnp.int32, (KPAD, 16), 1) == 9, 200.0, 0.0)

    def ext_cond(state):
        n, v, _ = state
        return (n < PRE_NMS_TOP_N) & (v > 0.0)

    def ext_body(state):
        n, v, rm = state
        rr = jnp.min(jnp.where(rm == v, flat2d, BIG))
        row = scores_ref[pl.ds(rr, 1), :]
        col = jnp.min(jnp.where(row == v, lane_c, BIG))
        regrow = regan_ref[pl.ds(rr, 1), :]
        meta = jnp.where(lane_8 == 0, v,
                         jnp.where(lane_8 == 1, col.astype(jnp.float32), 0.0))
        sel_ref[pl.ds(n, 1), :] = jnp.concatenate([regrow, meta], axis=1)
        new_row = jnp.where(lane_c == col, 0.0, row)
        scores_ref[pl.ds(rr, 1), :] = new_row
        rm_new = jnp.where(flat2d == rr, jnp.max(new_row), rm)
        return n + 1, jnp.max(rm_new), rm_new

    rm0 = rowmax_ref[...]
    jax.lax.while_loop(ext_cond, ext_body, (jnp.int32(0), jnp.max(rm0), rm0))

    wx, wy, ww, wh = WEIGHTS
    ra = sel_ref[...]
    widths = ra[:, 6:7] - ra[:, 4:5]
    heights = ra[:, 7:8] - ra[:, 5:6]
    ctr_x = ra[:, 4:5] + 0.5 * widths
    ctr_y = ra[:, 5:6] + 0.5 * heights
    dx = ra[:, 0:1] / wx
    dy = ra[:, 1:2] / wy
    dw = jnp.minimum(ra[:, 2:3] / ww, BBOX_XFORM_CLIP)
    dh = jnp.minimum(ra[:, 3:4] / wh, BBOX_XFORM_CLIP)
    pcx = dx * widths + ctr_x
    pcy = dy * heights + ctr_y
    pw = jnp.exp(dw) * widths
    ph = jnp.exp(dh) * heights
    x0 = pcx - 0.5 * pw
    y0 = pcy - 0.5 * ph
    x1 = pcx + 0.5 * pw
    y1 = pcy + 0.5 * ph
    boxes_ref[...] = jnp.concatenate([x0, y0, x1, y1], axis=1)

    off = ra[:, 9:10] * 10000.0
    ox0 = x0 + off
    oy0 = y0 + off
    ox1 = x1 + off
    oy1 = y1 + off
    area = (ox1 - ox0) * (oy1 - oy0)
    ox0r = ox0.reshape(1, KPAD)
    oy0r = oy0.reshape(1, KPAD)
    ox1r = ox1.reshape(1, KPAD)
    oy1r = oy1.reshape(1, KPAD)
    arear = area.reshape(1, KPAD)
    for q in range(KPAD // QROWS):
        sl = slice(q * QROWS, (q + 1) * QROWS)
        iw = jnp.maximum(jnp.minimum(ox1[sl], ox1r) - jnp.maximum(ox0[sl], ox0r), 0.0)
        ih = jnp.maximum(jnp.minimum(oy1[sl], oy1r) - jnp.maximum(oy0[sl], oy0r), 0.0)
        inter = iw * ih
        iou_ref[sl, :] = inter / (area[sl] + arear - inter + 1e-7)

    kiota = jax.lax.broadcasted_iota(jnp.int32, (1, KPAD), 1)

    def nms_body(i, keep):
        rows = iou_ref[pl.ds(i * NMSB, NMSB), :]
        for j in range(NMSB):
            idx = i * NMSB + j
            row = rows[j:j + 1, :]
            k_i = jnp.sum(jnp.where(kiota == idx, keep, 0.0))
            sup = (row > NMS_THRESH) & (kiota > idx) & (k_i > 0.5)
            keep = keep * (1.0 - sup.astype(jnp.float32))
        return keep

    keep = jax.lax.fori_loop(0, PRE_NMS_TOP_N // NMSB, nms_body,
                             jnp.ones((1, KPAD), jnp.float32))
    final = keep * sel_ref[:, 8:9].reshape(1, KPAD)

    def sel_body(k, fin):
        v = jnp.max(fin)
        am = jnp.min(jnp.where(fin == v, kiota, BIG))
        out_boxes_ref[pl.ds(k, 1), :] = boxes_ref[pl.ds(am, 1), :]
        out_scores_ref[pl.ds(k, 1), :] = v.reshape(1, 1)
        return jnp.where(kiota == am, -1.0, fin)

    jax.lax.fori_loop(0, POST_TOP_N, sel_body, final)


def kernel(box_cls, box_regression, iou_pred, anchors):
    regan = jnp.concatenate([box_regression, anchors], axis=1)
    boxes100, scores100 = pl.pallas_call(
        _paa_kernel,
        out_shape=[
            jax.ShapeDtypeStruct((POST_TOP_N, 4), jnp.float32),
            jax.ShapeDtypeStruct((POST_TOP_N, 1), jnp.float32),
        ],
        scratch_shapes=[
            pltpu.VMEM((N, C), jnp.float32),
            pltpu.VMEM((NCHUNK, CHUNK), jnp.float32),
            pltpu.VMEM((KPAD, 16), jnp.float32),
            pltpu.VMEM((KPAD, 4), jnp.float32),
            pltpu.VMEM((KPAD, KPAD), jnp.float32),
        ],
    )(box_cls, regan, iou_pred.reshape(NCHUNK, CHUNK))
    return jnp.concatenate([boxes100, scores100], axis=1)


# speedup vs baseline: 4.3148x; 1.0002x over previous
import jax
import jax.numpy as jnp
import numpy as np
from jax.experimental import pallas as pl
from jax.experimental.pallas import tpu as pltpu

N = 20000
C = 80
PRE_NMS_THRESH = 0.05
PRE_NMS_TOP_N = 1000
NMS_THRESH = 0.6
POST_TOP_N = 100
WEIGHTS = (10.0, 10.0, 5.0, 5.0)
BBOX_XFORM_CLIP = float(np.log(1000.0 / 16.0))

KPAD = 1024
NCHUNK = 40
CHUNK = N // NCHUNK
QROWS = 128
NMSB = 8
BIG = 1 << 30


def _paa_kernel(cls_ref, regan_ref, iou_in_ref,
                out_boxes_ref, out_scores_ref,
                scores_ref, rowmax_ref, sel_ref, boxes_ref, iou_ref):
    for k in range(NCHUNK):
        iou_col = iou_in_ref[k:k + 1, :].reshape(CHUNK, 1)
        s = jnp.sqrt(jax.nn.sigmoid(cls_ref[k * CHUNK:(k + 1) * CHUNK, :])
                     * jax.nn.sigmoid(iou_col))
        s = jnp.where(s > PRE_NMS_THRESH, s, 0.0)
        scores_ref[k * CHUNK:(k + 1) * CHUNK, :] = s
        rm = jnp.max(s, axis=1).reshape(1, CHUNK)
        rowmax_ref[k:k + 1, :] = rm

    lane_c = jax.lax.broadcasted_iota(jnp.int32, (1, C), 1)
    lane_8 = jax.lax.broadcasted_iota(jnp.int32, (1, 8), 1)
    flat2d = (jax.lax.broadcasted_iota(jnp.int32, (NCHUNK, CHUNK), 0) * CHUNK
              + jax.lax.broadcasted_iota(jnp.int32, (NCHUNK, CHUNK), 1))

    sel_ref[...] = jnp.where(
        jax.lax.broadcasted_iota(jnp.int32, (KPAD, 16), 1) == 9, 200.0, 0.0)

    def ext_cond(state):
        n, v, _ = state
        return (n < PRE_NMS_TOP_N) & (v > 0.0)

    def ext_body(state):
        n, v, rm = state
        rr = jnp.min(jnp.where(rm == v, flat2d, BIG))
        row = scores_ref[pl.ds(rr, 1), :]
        col = jnp.min(jnp.where(row == v, lane_c, BIG))
        regrow = regan_ref[pl.ds(rr, 1), :]
        meta = jnp.where(lane_8 == 0, v,
                         jnp.where(lane_8 == 1, col.astype(jnp.float32), 0.0))
        sel_ref[pl.ds(n, 1), :] = jnp.concatenate([regrow, meta], axis=1)
        new_row = jnp.where(lane_c == col, 0.0, row)
        scores_ref[pl.ds(rr, 1), :] = new_row
        rm_new = jnp.where(flat2d == rr, jnp.max(new_row), rm)
        return n + 1, jnp.max(rm_new), rm_new

    rm0 = rowmax_ref[...]
    jax.lax.while_loop(ext_cond, ext_body, (jnp.int32(0), jnp.max(rm0), rm0))

    wx, wy, ww, wh = WEIGHTS
    ra = sel_ref[...]
    widths = ra[:, 6:7] - ra[:, 4:5]
    heights = ra[:, 7:8] - ra[:, 5:6]
    ctr_x = ra[:, 4:5] + 0.5 * widths
    ctr_y = ra[:, 5:6] + 0.5 * heights
    dx = ra[:, 0:1] / wx
    dy = ra[:, 1:2] / wy
    dw = jnp.minimum(ra[:, 2:3] / ww, BBOX_XFORM_CLIP)
    dh = jnp.minimum(ra[:, 3:4] / wh, BBOX_XFORM_CLIP)
    pcx = dx * widths + ctr_x
    pcy = dy * heights + ctr_y
    pw = jnp.exp(dw) * widths
    ph = jnp.exp(dh) * heights
    x0 = pcx - 0.5 * pw
    y0 = pcy - 0.5 * ph
    x1 = pcx + 0.5 * pw
    y1 = pcy + 0.5 * ph
    boxes_ref[...] = jnp.concatenate([x0, y0, x1, y1], axis=1)

    off = ra[:, 9:10] * 10000.0
    ox0 = x0 + off
    oy0 = y0 + off
    ox1 = x1 + off
    oy1 = y1 + off
    area = (ox1 - ox0) * (oy1 - oy0)
    ox0r = ox0.reshape(1, KPAD)
    oy0r = oy0.reshape(1, KPAD)
    ox1r = ox1.reshape(1, KPAD)
    oy1r = oy1.reshape(1, KPAD)
    arear = area.reshape(1, KPAD)
    for q in range(KPAD // QROWS):
        sl = slice(q * QROWS, (q + 1) * QROWS)
        iw = jnp.maximum(jnp.minimum(ox1[sl], ox1r) - jnp.maximum(ox0[sl], ox0r), 0.0)
        ih = jnp.maximum(jnp.minimum(oy1[sl], oy1r) - jnp.maximum(oy0[sl], oy0r), 0.0)
        inter = iw * ih
        iou_ref[sl, :] = inter / (area[sl] + arear - inter + 1e-7)

    kiota = jax.lax.broadcasted_iota(jnp.int32, (1, KPAD), 1)

    def nms_body(i, keep):
        rows = iou_ref[pl.ds(i * NMSB, NMSB), :]
        for j in range(NMSB):
            idx = i * NMSB + j
            row = rows[j:j + 1, :]
            k_i = jnp.sum(jnp.where(kiota == idx, keep, 0.0))
            sup = (row > NMS_THRESH) & (kiota > idx) & (k_i > 0.5)
            keep = keep * (1.0 - sup.astype(jnp.float32))
        return keep

    keep = jax.lax.fori_loop(0, PRE_NMS_TOP_N // NMSB, nms_body,
                             jnp.ones((1, KPAD), jnp.float32))
    final = keep * sel_ref[:, 8:9].reshape(1, KPAD)

    def sel_body(k, fin):
        v = jnp.max(fin)
        am = jnp.min(jnp.where(fin == v, kiota, BIG))
        out_boxes_ref[pl.ds(k, 1), :] = boxes_ref[pl.ds(am, 1), :]
        out_scores_ref[pl.ds(k, 1), :] = v.reshape(1, 1)
        return jnp.where(kiota == am, -1.0, fin)

    jax.lax.fori_loop(0, POST_TOP_N, sel_body, final)


def kernel(box_cls, box_regression, iou_pred, anchors):
    regan = jnp.concatenate([box_regression, anchors], axis=1)
    boxes100, scores100 = pl.pallas_call(
        _paa_kernel,
        out_shape=[
            jax.ShapeDtypeStruct((POST_TOP_N, 4), jnp.float32),
            jax.ShapeDtypeStruct((POST_TOP_N, 1), jnp.float32),
        ],
        scratch_shapes=[
            pltpu.VMEM((N, C), jnp.float32),
            pltpu.VMEM((NCHUNK, CHUNK), jnp.float32),
            pltpu.VMEM((KPAD, 16), jnp.float32),
            pltpu.VMEM((KPAD, 4), jnp.float32),
            pltpu.VMEM((KPAD, KPAD), jnp.float32),
        ],
    )(box_cls, regan, iou_pred.reshape(NCHUNK, CHUNK))
    return jnp.concatenate([boxes100, scores100], axis=1)
